# baseline (speedup 1.0000x reference)
import jax
import jax.numpy as jnp
from jax import lax
from jax.experimental import pallas as pl
from jax.experimental.pallas import tpu as pltpu, tpu_sc as plsc

N = 4096
NC = 30
WIN = 100
D = 300
ATT_K = 25
MBLK = 64
WPAD = 128

NEG = -1e10

_SC_INFO = plsc.get_sparse_core_info()
_NWORKERS = _SC_INFO.num_cores * _SC_INFO.num_subcores


def _stage_a_body(emb_ref, tok_ref, tmask_ref, b1_ref, b2_ref,
                  ts_ref, s2_ref):
    b2 = b2_ref[...]
    b1 = b1_ref[...]
    ts_rows = []
    for m in range(MBLK):
        embc = jnp.concatenate([emb_ref[m] * b2, emb_ref[m] * b1], axis=0)
        s = jax.lax.dot_general(
            embc, tok_ref[m],
            dimension_numbers=(((1,), (1,)), ((), ())),
            preferred_element_type=jnp.float32)
        ts_rows.append(jnp.max(s[:NC], axis=0, keepdims=True))
        s2_ref[m] = s[NC:].astype(jnp.bfloat16)
    ts = jnp.concatenate(ts_rows, axis=0)
    ts = jnp.where(tmask_ref[...] > 0, ts, NEG)
    ts_ref[...] = jnp.concatenate(
        [ts, jnp.full((MBLK, WPAD - WIN), NEG, jnp.float32)], axis=1)


NCHUNKS = 4
CH = N // NCHUNKS
_ROWS_PER_W = CH // _NWORKERS


def _make_stage_a(c):
    base = c * (CH // MBLK)

    def fn(embeddings, token_embeddings, tmask, b1, b2):
        grid = (CH // MBLK,)
        return pl.pallas_call(
            _stage_a_body,
            grid=grid,
            in_specs=[
                pl.BlockSpec((MBLK, NC, D), lambda i: (base + i, 0, 0)),
                pl.BlockSpec((MBLK, WIN, D), lambda i: (base + i, 0, 0)),
                pl.BlockSpec((MBLK, WIN), lambda i: (base + i, 0)),
                pl.BlockSpec((1, D), lambda i: (0, 0)),
                pl.BlockSpec((1, D), lambda i: (0, 0)),
            ],
            out_specs=[
                pl.BlockSpec((MBLK, WPAD), lambda i: (i, 0)),
                pl.BlockSpec((MBLK, NC, WIN), lambda i: (i, 0, 0)),
            ],
            out_shape=[
                jax.ShapeDtypeStruct((CH, WPAD), jnp.float32),
                jax.ShapeDtypeStruct((CH, NC, WIN), jnp.bfloat16),
            ],
        )(embeddings, token_embeddings, tmask, b1, b2)

    return fn


_stage_a_chunks = [_make_stage_a(c) for c in range(NCHUNKS)]


def _tree_reduce(op, xs):
    xs = list(xs)
    while len(xs) > 1:
        nxt = [op(xs[2 * j], xs[2 * j + 1]) for j in range(len(xs) // 2)]
        if len(xs) % 2:
            nxt.append(xs[-1])
        xs = nxt
    return xs[0]


def _sc_group(in_v, out_v, g):
    base = lax.iota(jnp.int32, 16) * WPAD + g * (16 * WPAD)
    cols = [plsc.load_gather(in_v, [base + w]) for w in range(WPAD)]

    m0 = _tree_reduce(jnp.maximum, cols)

    def _extract(_, cur):
        masked = [jnp.where(c >= cur, -jnp.inf, c) for c in cols]
        return _tree_reduce(jnp.maximum, masked)

    thr = lax.fori_loop(0, ATT_K - 1, _extract, m0)

    es = [jnp.where(c >= thr, jnp.exp(c - m0), 0.0) for c in cols]
    ssum = _tree_reduce(jnp.add, es)
    for w in range(WPAD):
        plsc.store_scatter(out_v, [base + w], es[w] / ssum)
    return 0


def _sc_body(ts_hbm, probs_hbm, in_v, out_v):
    wid = lax.axis_index("s") * _SC_INFO.num_cores + lax.axis_index("c")
    base = wid * (_ROWS_PER_W * WPAD)
    pltpu.sync_copy(ts_hbm.at[pl.ds(base, _ROWS_PER_W * WPAD)], in_v)
    lax.fori_loop(0, _ROWS_PER_W // 16,
                  lambda g, c: _sc_group(in_v, out_v, g), 0)
    pltpu.sync_copy(out_v, probs_hbm.at[pl.ds(base, _ROWS_PER_W * WPAD)])


def _stage_b(ts):
    mesh = plsc.VectorSubcoreMesh(core_axis_name="c", subcore_axis_name="s")
    probs_flat = pl.kernel(
        _sc_body,
        out_type=jax.ShapeDtypeStruct((CH * WPAD,), jnp.float32),
        mesh=mesh,
        compiler_params=pltpu.CompilerParams(needs_layout_passes=False),
        scratch_types=[
            pltpu.VMEM((_ROWS_PER_W * WPAD,), jnp.float32),
            pltpu.VMEM((_ROWS_PER_W * WPAD,), jnp.float32),
        ],
    )(ts.reshape(CH * WPAD))
    return probs_flat.reshape(CH, WPAD)


def _stage_c_body(s2_ref, probs_ref, cmask_ref, out_ref):
    for m in range(MBLK):
        v = jax.lax.dot_general(
            probs_ref[m:m + 1, :WIN].astype(jnp.bfloat16), s2_ref[m],
            dimension_numbers=(((1,), (1,)), ((), ())),
            preferred_element_type=jnp.float32)
        out_ref[m, :] = jnp.where(cmask_ref[m, :] > 0, v[0], NEG)


def _make_stage_c(c):
    base = c * (CH // MBLK)

    def fn(s2, probs, cmask):
        grid = (CH // MBLK,)
        return pl.pallas_call(
            _stage_c_body,
            grid=grid,
            in_specs=[
                pl.BlockSpec((MBLK, NC, WIN), lambda i: (i, 0, 0)),
                pl.BlockSpec((MBLK, WPAD), lambda i: (i, 0)),
                pl.BlockSpec((MBLK, NC), lambda i: (base + i, 0)),
            ],
            out_specs=pl.BlockSpec((MBLK, NC), lambda i: (i, 0)),
            out_shape=jax.ShapeDtypeStruct((CH, NC), jnp.float32),
        )(s2, probs, cmask)

    return fn


_stage_c_chunks = [_make_stage_c(c) for c in range(NCHUNKS)]


@jax.jit
def _run(embeddings, cmask, token_embeddings, tmask, b1, b2):
    ab = [_stage_a_chunks[c](embeddings, token_embeddings, tmask, b1, b2)
          for c in range(NCHUNKS)]
    probs = [_stage_b(ts) for ts, _ in ab]
    outs = [_stage_c_chunks[c](ab[c][1], probs[c], cmask)
            for c in range(NCHUNKS)]
    return jnp.concatenate(outs, axis=0)


def kernel(n, embeddings, masks, token_embeddings, token_masks, B_diag1, B_diag2):
    del n
    cmask = masks.astype(jnp.int32)
    tmask = token_masks.astype(jnp.int32)
    b1 = B_diag1.reshape(1, D)
    b2 = B_diag2.reshape(1, D)
    return _run(embeddings, cmask, token_embeddings, tmask, b1, b2)


# speedup vs baseline: 1.0112x; 1.0112x over previous
import jax
import jax.numpy as jnp
from jax import lax
from jax.experimental import pallas as pl
from jax.experimental.pallas import tpu as pltpu, tpu_sc as plsc

N = 4096
NC = 30
WIN = 100
D = 300
ATT_K = 25
MBLK = 64
WPAD = 128

NEG = -1e10

_SC_INFO = plsc.get_sparse_core_info()
_NWORKERS = _SC_INFO.num_cores * _SC_INFO.num_subcores


def _stage_a_body(emb_ref, tok_ref, tmask_ref, b1_ref, b2_ref,
                  ts_ref, s2_ref):
    b2 = b2_ref[...]
    b1 = b1_ref[...]
    ts_rows = []
    for m in range(MBLK):
        embc = jnp.concatenate([emb_ref[m] * b2, emb_ref[m] * b1], axis=0)
        s = jax.lax.dot_general(
            embc, tok_ref[m],
            dimension_numbers=(((1,), (1,)), ((), ())),
            preferred_element_type=jnp.float32)
        ts_rows.append(jnp.max(s[:NC], axis=0, keepdims=True))
        s2_ref[m] = s[NC:].astype(jnp.bfloat16)
    ts = jnp.concatenate(ts_rows, axis=0)
    ts = jnp.where(tmask_ref[...] > 0, ts, NEG)
    ts_ref[...] = jnp.concatenate(
        [ts, jnp.full((MBLK, WPAD - WIN), NEG, jnp.float32)], axis=1)


NCHUNKS = 2
CH = N // NCHUNKS
_ROWS_PER_W = CH // _NWORKERS


def _make_stage_a(c):
    base = c * (CH // MBLK)

    def fn(embeddings, token_embeddings, tmask, b1, b2):
        grid = (CH // MBLK,)
        return pl.pallas_call(
            _stage_a_body,
            grid=grid,
            in_specs=[
                pl.BlockSpec((MBLK, NC, D), lambda i: (base + i, 0, 0)),
                pl.BlockSpec((MBLK, WIN, D), lambda i: (base + i, 0, 0)),
                pl.BlockSpec((MBLK, WIN), lambda i: (base + i, 0)),
                pl.BlockSpec((1, D), lambda i: (0, 0)),
                pl.BlockSpec((1, D), lambda i: (0, 0)),
            ],
            out_specs=[
                pl.BlockSpec((MBLK, WPAD), lambda i: (i, 0)),
                pl.BlockSpec((MBLK, NC, WIN), lambda i: (i, 0, 0)),
            ],
            out_shape=[
                jax.ShapeDtypeStruct((CH, WPAD), jnp.float32),
                jax.ShapeDtypeStruct((CH, NC, WIN), jnp.bfloat16),
            ],
        )(embeddings, token_embeddings, tmask, b1, b2)

    return fn


_stage_a_chunks = [_make_stage_a(c) for c in range(NCHUNKS)]


def _tree_reduce(op, xs):
    xs = list(xs)
    while len(xs) > 1:
        nxt = [op(xs[2 * j], xs[2 * j + 1]) for j in range(len(xs) // 2)]
        if len(xs) % 2:
            nxt.append(xs[-1])
        xs = nxt
    return xs[0]


def _sc_group(in_v, out_v, g):
    base = lax.iota(jnp.int32, 16) * WPAD + g * (16 * WPAD)
    cols = [plsc.load_gather(in_v, [base + w]) for w in range(WPAD)]

    m0 = _tree_reduce(jnp.maximum, cols)

    def _extract(_, cur):
        masked = [jnp.where(c >= cur, -jnp.inf, c) for c in cols]
        return _tree_reduce(jnp.maximum, masked)

    thr = lax.fori_loop(0, ATT_K - 1, _extract, m0)

    es = [jnp.where(c >= thr, jnp.exp(c - m0), 0.0) for c in cols]
    ssum = _tree_reduce(jnp.add, es)
    for w in range(WPAD):
        plsc.store_scatter(out_v, [base + w], es[w] / ssum)
    return 0


def _sc_body(ts_hbm, probs_hbm, in_v, out_v):
    wid = lax.axis_index("s") * _SC_INFO.num_cores + lax.axis_index("c")
    base = wid * (_ROWS_PER_W * WPAD)
    pltpu.sync_copy(ts_hbm.at[pl.ds(base, _ROWS_PER_W * WPAD)], in_v)
    lax.fori_loop(0, _ROWS_PER_W // 16,
                  lambda g, c: _sc_group(in_v, out_v, g), 0)
    pltpu.sync_copy(out_v, probs_hbm.at[pl.ds(base, _ROWS_PER_W * WPAD)])


def _stage_b(ts):
    mesh = plsc.VectorSubcoreMesh(core_axis_name="c", subcore_axis_name="s")
    probs_flat = pl.kernel(
        _sc_body,
        out_type=jax.ShapeDtypeStruct((CH * WPAD,), jnp.float32),
        mesh=mesh,
        compiler_params=pltpu.CompilerParams(needs_layout_passes=False),
        scratch_types=[
            pltpu.VMEM((_ROWS_PER_W * WPAD,), jnp.float32),
            pltpu.VMEM((_ROWS_PER_W * WPAD,), jnp.float32),
        ],
    )(ts.reshape(CH * WPAD))
    return probs_flat.reshape(CH, WPAD)


def _stage_c_body(s2_ref, probs_ref, cmask_ref, out_ref):
    for m in range(MBLK):
        v = jax.lax.dot_general(
            probs_ref[m:m + 1, :WIN].astype(jnp.bfloat16), s2_ref[m],
            dimension_numbers=(((1,), (1,)), ((), ())),
            preferred_element_type=jnp.float32)
        out_ref[m, :] = jnp.where(cmask_ref[m, :] > 0, v[0], NEG)


def _make_stage_c(c):
    base = c * (CH // MBLK)

    def fn(s2, probs, cmask):
        grid = (CH // MBLK,)
        return pl.pallas_call(
            _stage_c_body,
            grid=grid,
            in_specs=[
                pl.BlockSpec((MBLK, NC, WIN), lambda i: (i, 0, 0)),
                pl.BlockSpec((MBLK, WPAD), lambda i: (i, 0)),
                pl.BlockSpec((MBLK, NC), lambda i: (base + i, 0)),
            ],
            out_specs=pl.BlockSpec((MBLK, NC), lambda i: (i, 0)),
            out_shape=jax.ShapeDtypeStruct((CH, NC), jnp.float32),
        )(s2, probs, cmask)

    return fn


_stage_c_chunks = [_make_stage_c(c) for c in range(NCHUNKS)]


@jax.jit
def _run(embeddings, cmask, token_embeddings, tmask, b1, b2):
    ab = [_stage_a_chunks[c](embeddings, token_embeddings, tmask, b1, b2)
          for c in range(NCHUNKS)]
    probs = [_stage_b(ts) for ts, _ in ab]
    outs = [_stage_c_chunks[c](ab[c][1], probs[c], cmask)
            for c in range(NCHUNKS)]
    return jnp.concatenate(outs, axis=0)


def kernel(n, embeddings, masks, token_embeddings, token_masks, B_diag1, B_diag2):
    del n
    cmask = masks.astype(jnp.int32)
    tmask = token_masks.astype(jnp.int32)
    b1 = B_diag1.reshape(1, D)
    b2 = B_diag2.reshape(1, D)
    return _run(embeddings, cmask, token_embeddings, tmask, b1, b2)
